# baseline (device time: 173916 ns/iter reference)
import jax
import jax.numpy as jnp
from jax import lax
from jax.experimental import pallas as pl
from jax.experimental.pallas import tpu as pltpu

N_DEV = 4
N_TOK = 2048
D = 512
H = 1024
N_EXP = 16
E_LOCAL = N_EXP // N_DEV
CHUNK = N_TOK // N_DEV


def kernel(x, router_W, route_idx, expert_W, shared_W):
    def body(x_ref, router_ref, idx_ref, ew_ref, sw_ref, out_ref,
             comm_ref, rs_send_sems, rs_recv_sems, ag_send_sems, ag_recv_sems):
        my_pos = lax.axis_index("i")
        left = lax.rem(my_pos + N_DEV - 1, N_DEV)
        right = lax.rem(my_pos + 1, N_DEV)

        barrier_sem = pltpu.get_barrier_semaphore()
        for nbr in (left, right):
            pl.semaphore_signal(
                barrier_sem, inc=1,
                device_id=(nbr,), device_id_type=pl.DeviceIdType.MESH,
            )
        pl.semaphore_wait(barrier_sem, 2)

        xv = x_ref[:, :]
        scores = jnp.dot(xv, router_ref[:, :], preferred_element_type=jnp.float32)
        s_max = jnp.max(scores, axis=-1, keepdims=True)
        probs = jnp.exp(scores - s_max)
        probs = probs / jnp.sum(probs, axis=-1, keepdims=True)
        route = idx_ref[:, :]
        iota = lax.broadcasted_iota(jnp.int32, (N_TOK, N_EXP), 1)
        onehot = (iota == route).astype(jnp.float32)
        p_tok = jnp.sum(probs * onehot, axis=-1, keepdims=True)

        acc = jnp.zeros((N_TOK, H), jnp.float32)
        for le in range(E_LOCAL):
            ge = my_pos * E_LOCAL + le
            y = jnp.dot(xv, ew_ref[le], preferred_element_type=jnp.float32)
            gate = jnp.where(route == ge, p_tok, 0.0)
            acc = acc + gate * y
        out_ref[:, :] = acc

        for s in range(N_DEV - 1):
            cs = lax.rem(my_pos + N_DEV - s, N_DEV)
            cr = lax.rem(my_pos + 2 * N_DEV - s - 1, N_DEV)
            rdma = pltpu.make_async_remote_copy(
                src_ref=out_ref.at[pl.ds(cs * CHUNK, CHUNK), :],
                dst_ref=comm_ref.at[s],
                send_sem=rs_send_sems.at[s],
                recv_sem=rs_recv_sems.at[s],
                device_id=(right,),
                device_id_type=pl.DeviceIdType.MESH,
            )
            rdma.start()
            rdma.wait()
            out_ref[pl.ds(cr * CHUNK, CHUNK), :] = (
                out_ref[pl.ds(cr * CHUNK, CHUNK), :] + comm_ref[s]
            )

        own = lax.rem(my_pos + 1, N_DEV)
        row0 = own * CHUNK
        sh = jnp.dot(
            x_ref[pl.ds(row0, CHUNK), :], sw_ref[:, :],
            preferred_element_type=jnp.float32,
        )
        out_ref[pl.ds(row0, CHUNK), :] = out_ref[pl.ds(row0, CHUNK), :] + sh

        for s in range(N_DEV - 1):
            c = lax.rem(my_pos + N_DEV + 1 - s, N_DEV)
            rdma = pltpu.make_async_remote_copy(
                src_ref=out_ref.at[pl.ds(c * CHUNK, CHUNK), :],
                dst_ref=out_ref.at[pl.ds(c * CHUNK, CHUNK), :],
                send_sem=ag_send_sems.at[s],
                recv_sem=ag_recv_sems.at[s],
                device_id=(right,),
                device_id_type=pl.DeviceIdType.MESH,
            )
            rdma.start()
            rdma.wait()

    return pl.pallas_call(
        body,
        out_shape=jax.ShapeDtypeStruct((N_TOK, H), jnp.float32),
        in_specs=[
            pl.BlockSpec(memory_space=pltpu.VMEM),
            pl.BlockSpec(memory_space=pltpu.VMEM),
            pl.BlockSpec(memory_space=pltpu.VMEM),
            pl.BlockSpec(memory_space=pltpu.VMEM),
            pl.BlockSpec(memory_space=pltpu.VMEM),
        ],
        out_specs=pl.BlockSpec(memory_space=pltpu.VMEM),
        scratch_shapes=[
            pltpu.VMEM((N_DEV - 1, CHUNK, H), jnp.float32),
            pltpu.SemaphoreType.DMA((N_DEV - 1,)),
            pltpu.SemaphoreType.DMA((N_DEV - 1,)),
            pltpu.SemaphoreType.DMA((N_DEV - 1,)),
            pltpu.SemaphoreType.DMA((N_DEV - 1,)),
        ],
        compiler_params=pltpu.CompilerParams(collective_id=0),
    )(x, router_W, route_idx, expert_W, shared_W)


# device time: 106168 ns/iter; 1.6381x vs baseline; 1.6381x over previous
import jax
import jax.numpy as jnp
from jax import lax
from jax.experimental import pallas as pl
from jax.experimental.pallas import tpu as pltpu

N_DEV = 4
N_TOK = 2048
D = 512
H = 1024
N_EXP = 16
E_LOCAL = N_EXP // N_DEV
CHUNK = N_TOK // N_DEV
HALF = CHUNK // 2


def kernel(x, router_W, route_idx, expert_W, shared_W):
    def body(x_ref, router_ref, idx_ref, ew_ref, sw_ref, out_ref,
             r_comm, l_comm,
             r_rs_send, r_rs_recv, r_ag_send, r_ag_recv,
             l_rs_send, l_rs_recv, l_ag_send, l_ag_recv):
        my_pos = lax.axis_index("i")
        left = lax.rem(my_pos + N_DEV - 1, N_DEV)
        right = lax.rem(my_pos + 1, N_DEV)

        def r_rows(c):
            return pl.ds(c * CHUNK, HALF)

        def l_rows(c):
            return pl.ds(c * CHUNK + HALF, HALF)

        barrier_sem = pltpu.get_barrier_semaphore()
        for nbr in (left, right):
            pl.semaphore_signal(
                barrier_sem, inc=1,
                device_id=(nbr,), device_id_type=pl.DeviceIdType.MESH,
            )
        pl.semaphore_wait(barrier_sem, 2)

        xv = x_ref[:, :]
        scores = jnp.dot(xv, router_ref[:, :], preferred_element_type=jnp.float32)
        s_max = jnp.max(scores, axis=-1, keepdims=True)
        probs = jnp.exp(scores - s_max)
        probs = probs / jnp.sum(probs, axis=-1, keepdims=True)
        route = idx_ref[:, :]
        iota = lax.broadcasted_iota(jnp.int32, (N_TOK, N_EXP), 1)
        onehot = (iota == route).astype(jnp.float32)
        p_tok = jnp.sum(probs * onehot, axis=-1, keepdims=True)

        acc = jnp.zeros((N_TOK, H), jnp.float32)
        for le in range(E_LOCAL):
            ge = my_pos * E_LOCAL + le
            y = jnp.dot(xv, ew_ref[le], preferred_element_type=jnp.float32)
            gate = jnp.where(route == ge, p_tok, 0.0)
            acc = acc + gate * y
        out_ref[:, :] = acc

        for s in range(N_DEV - 1):
            cs_r = lax.rem(my_pos + N_DEV - s, N_DEV)
            cr_r = lax.rem(my_pos + 2 * N_DEV - s - 1, N_DEV)
            cs_l = lax.rem(my_pos + s, N_DEV)
            cr_l = lax.rem(my_pos + s + 1, N_DEV)
            r_rdma = pltpu.make_async_remote_copy(
                src_ref=out_ref.at[r_rows(cs_r), :],
                dst_ref=r_comm.at[s],
                send_sem=r_rs_send.at[s], recv_sem=r_rs_recv.at[s],
                device_id=(right,), device_id_type=pl.DeviceIdType.MESH,
            )
            l_rdma = pltpu.make_async_remote_copy(
                src_ref=out_ref.at[l_rows(cs_l), :],
                dst_ref=l_comm.at[s],
                send_sem=l_rs_send.at[s], recv_sem=l_rs_recv.at[s],
                device_id=(left,), device_id_type=pl.DeviceIdType.MESH,
            )
            r_rdma.start()
            l_rdma.start()
            r_rdma.wait()
            out_ref[r_rows(cr_r), :] = out_ref[r_rows(cr_r), :] + r_comm[s]
            l_rdma.wait()
            out_ref[l_rows(cr_l), :] = out_ref[l_rows(cr_l), :] + l_comm[s]

        own_r = lax.rem(my_pos + 1, N_DEV)
        own_l = lax.rem(my_pos + N_DEV - 1, N_DEV)
        sh_r = jnp.dot(x_ref[r_rows(own_r), :], sw_ref[:, :],
                       preferred_element_type=jnp.float32)
        out_ref[r_rows(own_r), :] = out_ref[r_rows(own_r), :] + sh_r
        sh_l = jnp.dot(x_ref[l_rows(own_l), :], sw_ref[:, :],
                       preferred_element_type=jnp.float32)
        out_ref[l_rows(own_l), :] = out_ref[l_rows(own_l), :] + sh_l

        for s in range(N_DEV - 1):
            c_r = lax.rem(my_pos + N_DEV + 1 - s, N_DEV)
            c_l = lax.rem(my_pos + N_DEV - 1 + s, N_DEV)
            r_rdma = pltpu.make_async_remote_copy(
                src_ref=out_ref.at[r_rows(c_r), :],
                dst_ref=out_ref.at[r_rows(c_r), :],
                send_sem=r_ag_send.at[s], recv_sem=r_ag_recv.at[s],
                device_id=(right,), device_id_type=pl.DeviceIdType.MESH,
            )
            l_rdma = pltpu.make_async_remote_copy(
                src_ref=out_ref.at[l_rows(c_l), :],
                dst_ref=out_ref.at[l_rows(c_l), :],
                send_sem=l_ag_send.at[s], recv_sem=l_ag_recv.at[s],
                device_id=(left,), device_id_type=pl.DeviceIdType.MESH,
            )
            r_rdma.start()
            l_rdma.start()
            r_rdma.wait()
            l_rdma.wait()

    return pl.pallas_call(
        body,
        out_shape=jax.ShapeDtypeStruct((N_TOK, H), jnp.float32),
        in_specs=[
            pl.BlockSpec(memory_space=pltpu.VMEM),
            pl.BlockSpec(memory_space=pltpu.VMEM),
            pl.BlockSpec(memory_space=pltpu.VMEM),
            pl.BlockSpec(memory_space=pltpu.VMEM),
            pl.BlockSpec(memory_space=pltpu.VMEM),
        ],
        out_specs=pl.BlockSpec(memory_space=pltpu.VMEM),
        scratch_shapes=[
            pltpu.VMEM((N_DEV - 1, HALF, H), jnp.float32),
            pltpu.VMEM((N_DEV - 1, HALF, H), jnp.float32),
            pltpu.SemaphoreType.DMA((N_DEV - 1,)),
            pltpu.SemaphoreType.DMA((N_DEV - 1,)),
            pltpu.SemaphoreType.DMA((N_DEV - 1,)),
            pltpu.SemaphoreType.DMA((N_DEV - 1,)),
            pltpu.SemaphoreType.DMA((N_DEV - 1,)),
            pltpu.SemaphoreType.DMA((N_DEV - 1,)),
            pltpu.SemaphoreType.DMA((N_DEV - 1,)),
            pltpu.SemaphoreType.DMA((N_DEV - 1,)),
        ],
        compiler_params=pltpu.CompilerParams(collective_id=0),
    )(x, router_W, route_idx, expert_W, shared_W)


# device time: 65674 ns/iter; 2.6482x vs baseline; 1.6166x over previous
import jax
import jax.numpy as jnp
from jax import lax
from jax.experimental import pallas as pl
from jax.experimental.pallas import tpu as pltpu

N_DEV = 4
N_TOK = 2048
D = 512
H = 1024
N_EXP = 16
E_LOCAL = N_EXP // N_DEV
CHUNK = N_TOK // N_DEV
HALF = CHUNK // 2
N_HOP = N_DEV - 1


def kernel(x, router_W, route_idx, expert_W, shared_W):
    def body(x_ref, router_ref, idx_ref, ew_ref, sw_ref, out_ref,
             gate_ref,
             r_rs_sbuf, r_rs_rbuf, r_ag_buf, r_own_buf,
             l_rs_sbuf, l_rs_rbuf, l_ag_buf, l_own_buf,
             r_rs_send, r_rs_recv, r_ag_send, r_ag_recv,
             l_rs_send, l_rs_recv, l_ag_send, l_ag_recv):
        my_pos = lax.axis_index("i")
        left = lax.rem(my_pos + N_DEV - 1, N_DEV)
        right = lax.rem(my_pos + 1, N_DEV)

        def r_rows(c):
            return pl.ds(c * CHUNK, HALF)

        def l_rows(c):
            return pl.ds(c * CHUNK + HALF, HALF)

        barrier_sem = pltpu.get_barrier_semaphore()
        for nbr in (left, right):
            pl.semaphore_signal(
                barrier_sem, inc=1,
                device_id=(nbr,), device_id_type=pl.DeviceIdType.MESH,
            )
        pl.semaphore_wait(barrier_sem, 2)

        xv = x_ref[:, :]
        scores = jnp.dot(xv, router_ref[:, :], preferred_element_type=jnp.float32)
        s_max = jnp.max(scores, axis=-1, keepdims=True)
        probs = jnp.exp(scores - s_max)
        probs = probs / jnp.sum(probs, axis=-1, keepdims=True)
        route = idx_ref[:, :]
        iota = lax.broadcasted_iota(jnp.int32, (N_TOK, N_EXP), 1)
        onehot = (iota == route).astype(jnp.float32)
        p_tok = jnp.sum(probs * onehot, axis=-1, keepdims=True)
        for le in range(E_LOCAL):
            ge = my_pos * E_LOCAL + le
            gate_ref[:, le:le + 1] = jnp.where(route == ge, p_tok, 0.0)

        def compute_half(row0, add_shared):
            x_c = x_ref[pl.ds(row0, HALF), :]
            if add_shared:
                acc = jnp.dot(x_c, sw_ref[:, :],
                              preferred_element_type=jnp.float32)
            else:
                acc = jnp.zeros((HALF, H), jnp.float32)
            for le in range(E_LOCAL):
                y = jnp.dot(x_c, ew_ref[le],
                            preferred_element_type=jnp.float32)
                acc = acc + gate_ref[pl.ds(row0, HALF), le:le + 1] * y
            return acc

        c0 = my_pos
        acc_r = compute_half(c0 * CHUNK, False)
        out_ref[r_rows(c0), :] = acc_r
        r_rs_sbuf[0] = acc_r.astype(jnp.bfloat16)
        r_h0 = pltpu.make_async_remote_copy(
            src_ref=r_rs_sbuf.at[0], dst_ref=r_rs_rbuf.at[0],
            send_sem=r_rs_send.at[0], recv_sem=r_rs_recv.at[0],
            device_id=(right,), device_id_type=pl.DeviceIdType.MESH,
        )
        r_h0.start()
        acc_l = compute_half(c0 * CHUNK + HALF, False)
        out_ref[l_rows(c0), :] = acc_l
        l_rs_sbuf[0] = acc_l.astype(jnp.bfloat16)
        l_h0 = pltpu.make_async_remote_copy(
            src_ref=l_rs_sbuf.at[0], dst_ref=l_rs_rbuf.at[0],
            send_sem=l_rs_send.at[0], recv_sem=l_rs_recv.at[0],
            device_id=(left,), device_id_type=pl.DeviceIdType.MESH,
        )
        l_h0.start()

        r_prev, l_prev = r_h0, l_h0
        for s in range(N_HOP):
            cr_r = lax.rem(my_pos + 2 * N_DEV - s - 1, N_DEV)
            cr_l = lax.rem(my_pos + s + 1, N_DEV)
            last = s == N_HOP - 1
            acc_r = compute_half(cr_r * CHUNK, last)
            out_ref[r_rows(cr_r), :] = acc_r
            acc_l = compute_half(cr_l * CHUNK + HALF, last)
            out_ref[l_rows(cr_l), :] = acc_l

            r_prev.wait()
            acc_r = acc_r + r_rs_rbuf[s].astype(jnp.float32)
            out_ref[r_rows(cr_r), :] = acc_r
            if not last:
                r_rs_sbuf[s + 1] = acc_r.astype(jnp.bfloat16)
                r_prev = pltpu.make_async_remote_copy(
                    src_ref=r_rs_sbuf.at[s + 1], dst_ref=r_rs_rbuf.at[s + 1],
                    send_sem=r_rs_send.at[s + 1], recv_sem=r_rs_recv.at[s + 1],
                    device_id=(right,), device_id_type=pl.DeviceIdType.MESH,
                )
                r_prev.start()
            else:
                r_own_buf[:, :] = acc_r.astype(jnp.bfloat16)

            l_prev.wait()
            acc_l = acc_l + l_rs_rbuf[s].astype(jnp.float32)
            out_ref[l_rows(cr_l), :] = acc_l
            if not last:
                l_rs_sbuf[s + 1] = acc_l.astype(jnp.bfloat16)
                l_prev = pltpu.make_async_remote_copy(
                    src_ref=l_rs_sbuf.at[s + 1], dst_ref=l_rs_rbuf.at[s + 1],
                    send_sem=l_rs_send.at[s + 1], recv_sem=l_rs_recv.at[s + 1],
                    device_id=(left,), device_id_type=pl.DeviceIdType.MESH,
                )
                l_prev.start()
            else:
                l_own_buf[:, :] = acc_l.astype(jnp.bfloat16)

        def ag_desc(src, dst, ssem, rsem, dev):
            return pltpu.make_async_remote_copy(
                src_ref=src, dst_ref=dst, send_sem=ssem, recv_sem=rsem,
                device_id=(dev,), device_id_type=pl.DeviceIdType.MESH,
            )

        r_ag = ag_desc(r_own_buf, r_ag_buf.at[0],
                       r_ag_send.at[0], r_ag_recv.at[0], right)
        r_ag.start()
        l_ag = ag_desc(l_own_buf, l_ag_buf.at[0],
                       l_ag_send.at[0], l_ag_recv.at[0], left)
        l_ag.start()
        for s in range(N_HOP):
            c_r = lax.rem(my_pos + N_DEV - s, N_DEV)
            c_l = lax.rem(my_pos + s, N_DEV)
            r_ag.wait()
            if s < N_HOP - 1:
                r_ag = ag_desc(r_ag_buf.at[s], r_ag_buf.at[s + 1],
                               r_ag_send.at[s + 1], r_ag_recv.at[s + 1], right)
                r_ag.start()
            out_ref[r_rows(c_r), :] = r_ag_buf[s].astype(jnp.float32)
            l_ag.wait()
            if s < N_HOP - 1:
                l_ag = ag_desc(l_ag_buf.at[s], l_ag_buf.at[s + 1],
                               l_ag_send.at[s + 1], l_ag_recv.at[s + 1], left)
                l_ag.start()
            out_ref[l_rows(c_l), :] = l_ag_buf[s].astype(jnp.float32)

    dma3 = pltpu.SemaphoreType.DMA((N_HOP,))
    return pl.pallas_call(
        body,
        out_shape=jax.ShapeDtypeStruct((N_TOK, H), jnp.float32),
        in_specs=[
            pl.BlockSpec(memory_space=pltpu.VMEM),
            pl.BlockSpec(memory_space=pltpu.VMEM),
            pl.BlockSpec(memory_space=pltpu.VMEM),
            pl.BlockSpec(memory_space=pltpu.VMEM),
            pl.BlockSpec(memory_space=pltpu.VMEM),
        ],
        out_specs=pl.BlockSpec(memory_space=pltpu.VMEM),
        scratch_shapes=[
            pltpu.VMEM((N_TOK, E_LOCAL), jnp.float32),
            pltpu.VMEM((N_HOP, HALF, H), jnp.bfloat16),
            pltpu.VMEM((N_HOP, HALF, H), jnp.bfloat16),
            pltpu.VMEM((N_HOP, HALF, H), jnp.bfloat16),
            pltpu.VMEM((HALF, H), jnp.bfloat16),
            pltpu.VMEM((N_HOP, HALF, H), jnp.bfloat16),
            pltpu.VMEM((N_HOP, HALF, H), jnp.bfloat16),
            pltpu.VMEM((N_HOP, HALF, H), jnp.bfloat16),
            pltpu.VMEM((HALF, H), jnp.bfloat16),
            dma3, dma3, dma3, dma3,
            dma3, dma3, dma3, dma3,
        ],
        compiler_params=pltpu.CompilerParams(collective_id=0),
    )(x, router_W, route_idx, expert_W, shared_W)


# device time: 65604 ns/iter; 2.6510x vs baseline; 1.0011x over previous
import jax
import jax.numpy as jnp
from jax import lax
from jax.experimental import pallas as pl
from jax.experimental.pallas import tpu as pltpu

N_DEV = 4
N_TOK = 2048
D = 512
H = 1024
N_EXP = 16
E_LOCAL = N_EXP // N_DEV
CHUNK = N_TOK // N_DEV
HALF = CHUNK // 2
N_HOP = N_DEV - 1


def kernel(x, router_W, route_idx, expert_W, shared_W):
    def body(x_ref, router_ref, idx_ref, ew_ref, sw_ref, out_ref,
             gate_ref, ew_bf, sw_bf,
             r_rs_sbuf, r_rs_rbuf, r_ag_buf, r_own_buf,
             l_rs_sbuf, l_rs_rbuf, l_ag_buf, l_own_buf,
             r_rs_send, r_rs_recv, r_ag_send, r_ag_recv,
             l_rs_send, l_rs_recv, l_ag_send, l_ag_recv):
        my_pos = lax.axis_index("i")
        left = lax.rem(my_pos + N_DEV - 1, N_DEV)
        right = lax.rem(my_pos + 1, N_DEV)

        def r_rows(c):
            return pl.ds(c * CHUNK, HALF)

        def l_rows(c):
            return pl.ds(c * CHUNK + HALF, HALF)

        barrier_sem = pltpu.get_barrier_semaphore()
        for nbr in (left, right):
            pl.semaphore_signal(
                barrier_sem, inc=1,
                device_id=(nbr,), device_id_type=pl.DeviceIdType.MESH,
            )
        pl.semaphore_wait(barrier_sem, 2)

        xv = x_ref[:, :]
        scores = jnp.dot(xv, router_ref[:, :], preferred_element_type=jnp.float32)
        s_max = jnp.max(scores, axis=-1, keepdims=True)
        probs = jnp.exp(scores - s_max)
        probs = probs / jnp.sum(probs, axis=-1, keepdims=True)
        route = idx_ref[:, :]
        iota = lax.broadcasted_iota(jnp.int32, (N_TOK, N_EXP), 1)
        onehot = (iota == route).astype(jnp.float32)
        p_tok = jnp.sum(probs * onehot, axis=-1, keepdims=True)
        for le in range(E_LOCAL):
            ge = my_pos * E_LOCAL + le
            gate_ref[:, le:le + 1] = jnp.where(route == ge, p_tok, 0.0)

        for le in range(E_LOCAL):
            ew_bf[le] = ew_ref[le].astype(jnp.bfloat16)
        sw_bf[:, :] = sw_ref[:, :].astype(jnp.bfloat16)

        def compute_half(row0, add_shared):
            x_c = x_ref[pl.ds(row0, HALF), :].astype(jnp.bfloat16)
            if add_shared:
                acc = jnp.dot(x_c, sw_bf[:, :],
                              preferred_element_type=jnp.float32)
            else:
                acc = jnp.zeros((HALF, H), jnp.float32)
            for le in range(E_LOCAL):
                y = jnp.dot(x_c, ew_bf[le],
                            preferred_element_type=jnp.float32)
                acc = acc + gate_ref[pl.ds(row0, HALF), le:le + 1] * y
            return acc

        c0 = my_pos
        acc_r = compute_half(c0 * CHUNK, False)
        r_rs_sbuf[0] = acc_r.astype(jnp.bfloat16)
        r_h0 = pltpu.make_async_remote_copy(
            src_ref=r_rs_sbuf.at[0], dst_ref=r_rs_rbuf.at[0],
            send_sem=r_rs_send.at[0], recv_sem=r_rs_recv.at[0],
            device_id=(right,), device_id_type=pl.DeviceIdType.MESH,
        )
        r_h0.start()
        acc_l = compute_half(c0 * CHUNK + HALF, False)
        l_rs_sbuf[0] = acc_l.astype(jnp.bfloat16)
        l_h0 = pltpu.make_async_remote_copy(
            src_ref=l_rs_sbuf.at[0], dst_ref=l_rs_rbuf.at[0],
            send_sem=l_rs_send.at[0], recv_sem=l_rs_recv.at[0],
            device_id=(left,), device_id_type=pl.DeviceIdType.MESH,
        )
        l_h0.start()

        r_prev, l_prev = r_h0, l_h0
        for s in range(N_HOP):
            cr_r = lax.rem(my_pos + 2 * N_DEV - s - 1, N_DEV)
            cr_l = lax.rem(my_pos + s + 1, N_DEV)
            last = s == N_HOP - 1
            acc_r = compute_half(cr_r * CHUNK, last)
            acc_l = compute_half(cr_l * CHUNK + HALF, last)

            r_prev.wait()
            acc_r = acc_r + r_rs_rbuf[s].astype(jnp.float32)
            if not last:
                r_rs_sbuf[s + 1] = acc_r.astype(jnp.bfloat16)
                r_prev = pltpu.make_async_remote_copy(
                    src_ref=r_rs_sbuf.at[s + 1], dst_ref=r_rs_rbuf.at[s + 1],
                    send_sem=r_rs_send.at[s + 1], recv_sem=r_rs_recv.at[s + 1],
                    device_id=(right,), device_id_type=pl.DeviceIdType.MESH,
                )
                r_prev.start()
            else:
                out_ref[r_rows(cr_r), :] = acc_r
                r_own_buf[:, :] = acc_r.astype(jnp.bfloat16)

            l_prev.wait()
            acc_l = acc_l + l_rs_rbuf[s].astype(jnp.float32)
            if not last:
                l_rs_sbuf[s + 1] = acc_l.astype(jnp.bfloat16)
                l_prev = pltpu.make_async_remote_copy(
                    src_ref=l_rs_sbuf.at[s + 1], dst_ref=l_rs_rbuf.at[s + 1],
                    send_sem=l_rs_send.at[s + 1], recv_sem=l_rs_recv.at[s + 1],
                    device_id=(left,), device_id_type=pl.DeviceIdType.MESH,
                )
                l_prev.start()
            else:
                out_ref[l_rows(cr_l), :] = acc_l
                l_own_buf[:, :] = acc_l.astype(jnp.bfloat16)

        def ag_desc(src, dst, ssem, rsem, dev):
            return pltpu.make_async_remote_copy(
                src_ref=src, dst_ref=dst, send_sem=ssem, recv_sem=rsem,
                device_id=(dev,), device_id_type=pl.DeviceIdType.MESH,
            )

        r_ag = ag_desc(r_own_buf, r_ag_buf.at[0],
                       r_ag_send.at[0], r_ag_recv.at[0], right)
        r_ag.start()
        l_ag = ag_desc(l_own_buf, l_ag_buf.at[0],
                       l_ag_send.at[0], l_ag_recv.at[0], left)
        l_ag.start()
        for s in range(N_HOP):
            c_r = lax.rem(my_pos + N_DEV - s, N_DEV)
            c_l = lax.rem(my_pos + s, N_DEV)
            r_ag.wait()
            if s < N_HOP - 1:
                r_ag = ag_desc(r_ag_buf.at[s], r_ag_buf.at[s + 1],
                               r_ag_send.at[s + 1], r_ag_recv.at[s + 1], right)
                r_ag.start()
            out_ref[r_rows(c_r), :] = r_ag_buf[s].astype(jnp.float32)
            l_ag.wait()
            if s < N_HOP - 1:
                l_ag = ag_desc(l_ag_buf.at[s], l_ag_buf.at[s + 1],
                               l_ag_send.at[s + 1], l_ag_recv.at[s + 1], left)
                l_ag.start()
            out_ref[l_rows(c_l), :] = l_ag_buf[s].astype(jnp.float32)

    dma3 = pltpu.SemaphoreType.DMA((N_HOP,))
    return pl.pallas_call(
        body,
        out_shape=jax.ShapeDtypeStruct((N_TOK, H), jnp.float32),
        in_specs=[
            pl.BlockSpec(memory_space=pltpu.VMEM),
            pl.BlockSpec(memory_space=pltpu.VMEM),
            pl.BlockSpec(memory_space=pltpu.VMEM),
            pl.BlockSpec(memory_space=pltpu.VMEM),
            pl.BlockSpec(memory_space=pltpu.VMEM),
        ],
        out_specs=pl.BlockSpec(memory_space=pltpu.VMEM),
        scratch_shapes=[
            pltpu.VMEM((N_TOK, E_LOCAL), jnp.float32),
            pltpu.VMEM((E_LOCAL, D, H), jnp.bfloat16),
            pltpu.VMEM((D, H), jnp.bfloat16),
            pltpu.VMEM((N_HOP, HALF, H), jnp.bfloat16),
            pltpu.VMEM((N_HOP, HALF, H), jnp.bfloat16),
            pltpu.VMEM((N_HOP, HALF, H), jnp.bfloat16),
            pltpu.VMEM((HALF, H), jnp.bfloat16),
            pltpu.VMEM((N_HOP, HALF, H), jnp.bfloat16),
            pltpu.VMEM((N_HOP, HALF, H), jnp.bfloat16),
            pltpu.VMEM((N_HOP, HALF, H), jnp.bfloat16),
            pltpu.VMEM((HALF, H), jnp.bfloat16),
            dma3, dma3, dma3, dma3,
            dma3, dma3, dma3, dma3,
        ],
        compiler_params=pltpu.CompilerParams(collective_id=0),
    )(x, router_W, route_idx, expert_W, shared_W)
